# baseline (speedup 1.0000x reference)
import jax
import jax.numpy as jnp
from jax.experimental import pallas as pl
from jax.experimental.pallas import tpu as pltpu

B, P, D = 4, 2048, 1024
CH = 256
N_S = P // CH
NBUF = 3
LOOK = 2
U = B * N_S


def _body(x_hbm, lut_hbm, out_hbm, lut_v, b0, b1, b2,
          si0, si1, si2, so0, so1, so2, slut):
    bufs = (b0, b1, b2)
    sin = (si0, si1, si2)
    sout = (so0, so1, so2)

    def pos(k):
        return k // N_S, k % N_S

    def start_in(k):
        b, s = pos(k)
        i = k % NBUF
        pltpu.make_async_copy(
            x_hbm.at[b, pl.ds(s * CH, CH)], bufs[i], sin[i]).start()

    def wait_in(k):
        i = k % NBUF
        pltpu.make_async_copy(
            x_hbm.at[0, pl.ds(0, CH)], bufs[i], sin[i]).wait()

    def start_out(k):
        b, s = pos(k)
        i = k % NBUF
        pltpu.make_async_copy(
            bufs[i], out_hbm.at[b, pl.ds(s * CH, CH)], sout[i]).start()

    def wait_out(k):
        i = k % NBUF
        pltpu.make_async_copy(
            bufs[i], out_hbm.at[0, pl.ds(0, CH)], sout[i]).wait()

    def start_lut(s):
        pltpu.make_async_copy(
            lut_hbm.at[pl.ds(s * CH, CH)],
            lut_v.at[pl.ds(s * CH, CH)], slut).start()

    def wait_lut(s):
        pltpu.make_async_copy(
            lut_hbm.at[pl.ds(0, CH)], lut_v.at[pl.ds(0, CH)], slut).wait()

    for k in range(LOOK):
        start_in(k)
        start_lut(k)

    for k in range(U):
        if k + LOOK < U:
            if k + LOOK >= NBUF:
                wait_out(k + LOOK - NBUF)
            start_in(k + LOOK)
            if k + LOOK < N_S:
                start_lut(k + LOOK)
        wait_in(k)
        b, s = pos(k)
        if b == 0:
            wait_lut(s)
        i = k % NBUF
        bufs[i][...] = bufs[i][...] + lut_v[pl.ds(s * CH, CH), :]
        start_out(k)

    for k in range(U - NBUF, U):
        wait_out(k)


def kernel(x, lut_weight):
    return pl.pallas_call(
        _body,
        in_specs=[
            pl.BlockSpec(memory_space=pl.ANY),
            pl.BlockSpec(memory_space=pl.ANY),
        ],
        out_specs=pl.BlockSpec(memory_space=pl.ANY),
        out_shape=jax.ShapeDtypeStruct((B, P, D), x.dtype),
        scratch_shapes=(
            [pltpu.VMEM((P, D), jnp.float32)]
            + [pltpu.VMEM((CH, D), jnp.float32)] * NBUF
            + [pltpu.SemaphoreType.DMA] * (2 * NBUF + 1)
        ),
    )(x, lut_weight)


# speedup vs baseline: 1.5553x; 1.5553x over previous
import jax
import jax.numpy as jnp
from jax.experimental import pallas as pl
from jax.experimental.pallas import tpu as pltpu

B, P, D = 4, 2048, 1024
CH = 256
N_S = P // CH
NBUF = 6
LOOK = 3
U = B * N_S


def _body(x_hbm, lut_hbm, out_hbm, lut_v, b0, b1, b2, b3, b4, b5,
          si0, si1, si2, si3, si4, si5, so0, so1, so2, so3, so4, so5, slut):
    bufs = (b0, b1, b2, b3, b4, b5)
    sin = (si0, si1, si2, si3, si4, si5)
    sout = (so0, so1, so2, so3, so4, so5)

    def pos(k):
        return k // N_S, k % N_S

    def start_in(k):
        b, s = pos(k)
        i = k % NBUF
        pltpu.make_async_copy(
            x_hbm.at[b, pl.ds(s * CH, CH)], bufs[i], sin[i]).start()

    def wait_in(k):
        i = k % NBUF
        pltpu.make_async_copy(
            x_hbm.at[0, pl.ds(0, CH)], bufs[i], sin[i]).wait()

    def start_out(k):
        b, s = pos(k)
        i = k % NBUF
        pltpu.make_async_copy(
            bufs[i], out_hbm.at[b, pl.ds(s * CH, CH)], sout[i]).start()

    def wait_out(k):
        i = k % NBUF
        pltpu.make_async_copy(
            bufs[i], out_hbm.at[0, pl.ds(0, CH)], sout[i]).wait()

    def start_lut(s):
        pltpu.make_async_copy(
            lut_hbm.at[pl.ds(s * CH, CH)],
            lut_v.at[pl.ds(s * CH, CH)], slut).start()

    def wait_lut(s):
        pltpu.make_async_copy(
            lut_hbm.at[pl.ds(0, CH)], lut_v.at[pl.ds(0, CH)], slut).wait()

    for k in range(LOOK):
        start_in(k)
        start_lut(k)

    for k in range(U):
        if k + LOOK < U:
            if k + LOOK >= NBUF:
                wait_out(k + LOOK - NBUF)
            start_in(k + LOOK)
            if k + LOOK < N_S:
                start_lut(k + LOOK)
        wait_in(k)
        b, s = pos(k)
        if b == 0:
            wait_lut(s)
        i = k % NBUF
        bufs[i][...] = bufs[i][...] + lut_v[pl.ds(s * CH, CH), :]
        start_out(k)

    for k in range(U - NBUF, U):
        wait_out(k)


def kernel(x, lut_weight):
    return pl.pallas_call(
        _body,
        in_specs=[
            pl.BlockSpec(memory_space=pl.ANY),
            pl.BlockSpec(memory_space=pl.ANY),
        ],
        out_specs=pl.BlockSpec(memory_space=pl.ANY),
        out_shape=jax.ShapeDtypeStruct((B, P, D), x.dtype),
        scratch_shapes=(
            [pltpu.VMEM((P, D), jnp.float32)]
            + [pltpu.VMEM((CH, D), jnp.float32)] * NBUF
            + [pltpu.SemaphoreType.DMA] * (2 * NBUF + 1)
        ),
    )(x, lut_weight)


# speedup vs baseline: 1.6058x; 1.0324x over previous
import jax
import jax.numpy as jnp
from jax.experimental import pallas as pl
from jax.experimental.pallas import tpu as pltpu

B, P, D = 4, 2048, 1024
CH = 512
N_S = P // CH
NBUF = 6
LOOK = 3
U = B * N_S


def _body(x_hbm, lut_hbm, out_hbm, lut_v, b0, b1, b2, b3, b4, b5,
          si0, si1, si2, si3, si4, si5, so0, so1, so2, so3, so4, so5, slut):
    bufs = (b0, b1, b2, b3, b4, b5)
    sin = (si0, si1, si2, si3, si4, si5)
    sout = (so0, so1, so2, so3, so4, so5)

    def pos(k):
        return k // N_S, k % N_S

    def start_in(k):
        b, s = pos(k)
        i = k % NBUF
        pltpu.make_async_copy(
            x_hbm.at[b, pl.ds(s * CH, CH)], bufs[i], sin[i]).start()

    def wait_in(k):
        i = k % NBUF
        pltpu.make_async_copy(
            x_hbm.at[0, pl.ds(0, CH)], bufs[i], sin[i]).wait()

    def start_out(k):
        b, s = pos(k)
        i = k % NBUF
        pltpu.make_async_copy(
            bufs[i], out_hbm.at[b, pl.ds(s * CH, CH)], sout[i]).start()

    def wait_out(k):
        i = k % NBUF
        pltpu.make_async_copy(
            bufs[i], out_hbm.at[0, pl.ds(0, CH)], sout[i]).wait()

    def start_lut(s):
        pltpu.make_async_copy(
            lut_hbm.at[pl.ds(s * CH, CH)],
            lut_v.at[pl.ds(s * CH, CH)], slut).start()

    def wait_lut(s):
        pltpu.make_async_copy(
            lut_hbm.at[pl.ds(0, CH)], lut_v.at[pl.ds(0, CH)], slut).wait()

    for k in range(LOOK):
        start_in(k)
        start_lut(k)

    for k in range(U):
        if k + LOOK < U:
            if k + LOOK >= NBUF:
                wait_out(k + LOOK - NBUF)
            start_in(k + LOOK)
            if k + LOOK < N_S:
                start_lut(k + LOOK)
        wait_in(k)
        b, s = pos(k)
        if b == 0:
            wait_lut(s)
        i = k % NBUF
        bufs[i][...] = bufs[i][...] + lut_v[pl.ds(s * CH, CH), :]
        start_out(k)

    for k in range(U - NBUF, U):
        wait_out(k)


def kernel(x, lut_weight):
    return pl.pallas_call(
        _body,
        in_specs=[
            pl.BlockSpec(memory_space=pl.ANY),
            pl.BlockSpec(memory_space=pl.ANY),
        ],
        out_specs=pl.BlockSpec(memory_space=pl.ANY),
        out_shape=jax.ShapeDtypeStruct((B, P, D), x.dtype),
        scratch_shapes=(
            [pltpu.VMEM((P, D), jnp.float32)]
            + [pltpu.VMEM((CH, D), jnp.float32)] * NBUF
            + [pltpu.SemaphoreType.DMA] * (2 * NBUF + 1)
        ),
    )(x, lut_weight)


# speedup vs baseline: 1.6086x; 1.0018x over previous
import jax
import jax.numpy as jnp
from jax.experimental import pallas as pl
from jax.experimental.pallas import tpu as pltpu

B, P, D = 4, 2048, 1024
CH = 512
N_S = P // CH
NBUF = 8
LOOK = 4
U = B * N_S


def _body(x_hbm, lut_hbm, out_hbm, lut_v, b0, b1, b2, b3, b4, b5, b6, b7,
          si0, si1, si2, si3, si4, si5, si6, si7,
          so0, so1, so2, so3, so4, so5, so6, so7, slut):
    bufs = (b0, b1, b2, b3, b4, b5, b6, b7)
    sin = (si0, si1, si2, si3, si4, si5, si6, si7)
    sout = (so0, so1, so2, so3, so4, so5, so6, so7)

    def pos(k):
        return k // N_S, k % N_S

    def start_in(k):
        b, s = pos(k)
        i = k % NBUF
        pltpu.make_async_copy(
            x_hbm.at[b, pl.ds(s * CH, CH)], bufs[i], sin[i]).start()

    def wait_in(k):
        i = k % NBUF
        pltpu.make_async_copy(
            x_hbm.at[0, pl.ds(0, CH)], bufs[i], sin[i]).wait()

    def start_out(k):
        b, s = pos(k)
        i = k % NBUF
        pltpu.make_async_copy(
            bufs[i], out_hbm.at[b, pl.ds(s * CH, CH)], sout[i]).start()

    def wait_out(k):
        i = k % NBUF
        pltpu.make_async_copy(
            bufs[i], out_hbm.at[0, pl.ds(0, CH)], sout[i]).wait()

    def start_lut(s):
        pltpu.make_async_copy(
            lut_hbm.at[pl.ds(s * CH, CH)],
            lut_v.at[pl.ds(s * CH, CH)], slut).start()

    def wait_lut(s):
        pltpu.make_async_copy(
            lut_hbm.at[pl.ds(0, CH)], lut_v.at[pl.ds(0, CH)], slut).wait()

    for k in range(LOOK):
        start_in(k)
        start_lut(k)

    for k in range(U):
        if k + LOOK < U:
            if k + LOOK >= NBUF:
                wait_out(k + LOOK - NBUF)
            start_in(k + LOOK)
            if k + LOOK < N_S:
                start_lut(k + LOOK)
        wait_in(k)
        b, s = pos(k)
        if b == 0:
            wait_lut(s)
        i = k % NBUF
        bufs[i][...] = bufs[i][...] + lut_v[pl.ds(s * CH, CH), :]
        start_out(k)

    for k in range(U - NBUF, U):
        wait_out(k)


def kernel(x, lut_weight):
    return pl.pallas_call(
        _body,
        in_specs=[
            pl.BlockSpec(memory_space=pl.ANY),
            pl.BlockSpec(memory_space=pl.ANY),
        ],
        out_specs=pl.BlockSpec(memory_space=pl.ANY),
        out_shape=jax.ShapeDtypeStruct((B, P, D), x.dtype),
        scratch_shapes=(
            [pltpu.VMEM((P, D), jnp.float32)]
            + [pltpu.VMEM((CH, D), jnp.float32)] * NBUF
            + [pltpu.SemaphoreType.DMA] * (2 * NBUF + 1)
        ),
    )(x, lut_weight)


# speedup vs baseline: 1.6367x; 1.0174x over previous
import jax
import jax.numpy as jnp
from jax.experimental import pallas as pl

BLK_P = 2048


def _add_body(x_ref, lut_ref, o_ref):
    o_ref[...] = x_ref[...] + lut_ref[...]


def kernel(x, lut_weight):
    B, P, D = x.shape
    grid = (P // BLK_P, B)
    return pl.pallas_call(
        _add_body,
        grid=grid,
        in_specs=[
            pl.BlockSpec((1, BLK_P, D), lambda i, j: (j, i, 0)),
            pl.BlockSpec((BLK_P, D), lambda i, j: (i, 0)),
        ],
        out_specs=pl.BlockSpec((1, BLK_P, D), lambda i, j: (j, i, 0)),
        out_shape=jax.ShapeDtypeStruct((B, P, D), x.dtype),
    )(x, lut_weight)
